# baseline (speedup 1.0000x reference)
import functools

import jax
import jax.numpy as jnp
from jax import lax
from jax.experimental import pallas as pl
from jax.experimental.pallas import tpu as pltpu
from jax.experimental.pallas import tpu_sc as plsc

NUM_ITEMS = 1000000
D = 64
B = 4096
L = 16

_info = plsc.get_sparse_core_info()
NC, NS = _info.num_cores, _info.num_subcores
NW = NC * NS
BPW = B // NW

_mesh = plsc.VectorSubcoreMesh(core_axis_name="c", subcore_axis_name="s")


def _sigmoid_v(x):
    return 1.0 / (1.0 + jnp.exp(-x))


def _hsum_splat(x):
    lanes = jnp.arange(L, dtype=jnp.int32)
    for stride in (8, 4, 2, 1):
        x = x + x.at[lanes ^ stride].get(mode="promise_in_bounds")
    return x


@functools.partial(
    pl.kernel,
    mesh=_mesh,
    compiler_params=pltpu.CompilerParams(use_tc_tiling_on_sc=False),
    out_type=jax.ShapeDtypeStruct((NW, D), jnp.float32),
    scratch_types=[
        pltpu.VMEM((BPW,), jnp.int32),
        pltpu.VMEM((BPW, D), jnp.float32),
        pltpu.VMEM((BPW, D), jnp.float32),
        pltpu.VMEM((D,), jnp.float32),
        pltpu.VMEM((L,), jnp.float32),
        pltpu.VMEM((D,), jnp.float32),
        pltpu.SemaphoreType.DMA,
        pltpu.SemaphoreType.DMA,
    ],
)
def _others_partials(idx_hbm, gmf_hbm, sdae_hbm, w_hbm, b_hbm, part_hbm,
                     idx_v, gmf_v, sdae_v, w_v, b_v, acc_v, sem1, sem2):
    wid = lax.axis_index("s") * NC + lax.axis_index("c")
    base = wid * BPW
    pltpu.sync_copy(idx_hbm.at[pl.ds(base, BPW)], idx_v)
    cp1 = pltpu.async_copy(gmf_hbm.at[idx_v], gmf_v, sem1)
    cp2 = pltpu.async_copy(sdae_hbm.at[idx_v], sdae_v, sem2)
    pltpu.sync_copy(w_hbm, w_v)
    pltpu.sync_copy(b_hbm, b_v)
    cp1.wait()
    cp2.wait()

    w = [w_v[pl.ds(k * L, L)] for k in range(D // L)]
    bvec = b_v[...]

    def row(j, acc):
        g = [gmf_v[j, pl.ds(k * L, L)] for k in range(D // L)]
        t = g[0] * w[0]
        for k in range(1, D // L):
            t = t + g[k] * w[k]
        zv = _hsum_splat(t) + bvec
        dv = _sigmoid_v(zv)
        out = []
        for k in range(D // L):
            sd = sdae_v[j, pl.ds(k * L, L)]
            out.append(acc[k] + g[k] + dv * (sd - g[k]))
        return tuple(out)

    zero = jnp.zeros((L,), jnp.float32)
    acc = lax.fori_loop(0, BPW, row, (zero,) * (D // L))
    for k in range(D // L):
        acc_v[pl.ds(k * L, L)] = acc[k]
    pltpu.sync_copy(acc_v, part_hbm.at[wid])


@functools.partial(
    pl.kernel,
    mesh=_mesh,
    compiler_params=pltpu.CompilerParams(use_tc_tiling_on_sc=False),
    out_type=jax.ShapeDtypeStruct((B,), jnp.float32),
    scratch_types=[
        pltpu.VMEM((BPW,), jnp.int32),
        pltpu.VMEM((BPW, D), jnp.float32),
        pltpu.VMEM((BPW, D), jnp.float32),
        pltpu.VMEM((D,), jnp.float32),
        pltpu.VMEM((L,), jnp.float32),
        pltpu.VMEM((NW, D), jnp.float32),
        pltpu.VMEM((BPW,), jnp.float32),
        pltpu.SemaphoreType.DMA,
        pltpu.SemaphoreType.DMA,
    ],
)
def _target_dots(idx_hbm, gmf_hbm, sdae_hbm, w_hbm, b_hbm, part_hbm, out_hbm,
                 idx_v, gmf_v, sdae_v, w_v, b_v, p_v, out_v, sem1, sem2):
    wid = lax.axis_index("s") * NC + lax.axis_index("c")
    base = wid * BPW
    pltpu.sync_copy(idx_hbm.at[pl.ds(base, BPW)], idx_v)
    cp1 = pltpu.async_copy(gmf_hbm.at[idx_v], gmf_v, sem1)
    cp2 = pltpu.async_copy(sdae_hbm.at[idx_v], sdae_v, sem2)
    pltpu.sync_copy(w_hbm, w_v)
    pltpu.sync_copy(b_hbm, b_v)
    pltpu.sync_copy(part_hbm, p_v)

    w = [w_v[pl.ds(k * L, L)] for k in range(D // L)]
    bvec = b_v[...]
    s = []
    for k in range(D // L):
        sk = p_v[0, pl.ds(k * L, L)]
        for n in range(1, NW):
            sk = sk + p_v[n, pl.ds(k * L, L)]
        s.append(sk)

    cp1.wait()
    cp2.wait()

    lanes = lax.iota(jnp.int32, L)

    def chunk(ci, carry):
        dotv = jnp.zeros((L,), jnp.float32)
        for jj in range(L):
            j = ci * L + jj
            g = [gmf_v[j, pl.ds(k * L, L)] for k in range(D // L)]
            t = g[0] * w[0]
            for k in range(1, D // L):
                t = t + g[k] * w[k]
            zv = _hsum_splat(t) + bvec
            dv = _sigmoid_v(zv)
            u = None
            for k in range(D // L):
                sd = sdae_v[j, pl.ds(k * L, L)]
                blend = g[k] + dv * (sd - g[k])
                u = blend * s[k] if u is None else u + blend * s[k]
            dotv = jnp.where(lanes == jj, _hsum_splat(u), dotv)
        out_v[pl.ds(ci * L, L)] = _sigmoid_v(dotv)
        return carry

    lax.fori_loop(0, BPW // L, chunk, 0)
    pltpu.sync_copy(out_v, out_hbm.at[pl.ds(base, BPW)])


def kernel(item_indices1, item_indices2, table_gmf, table_sdae, delta_W, delta_b):
    w = delta_W.reshape(D).astype(jnp.float32)
    b = jnp.broadcast_to(delta_b.reshape(1), (L,)).astype(jnp.float32)
    partials = _others_partials(item_indices2.astype(jnp.int32),
                                table_gmf, table_sdae, w, b)
    dots = _target_dots(item_indices1.astype(jnp.int32),
                        table_gmf, table_sdae, w, b, partials)
    return dots.reshape(B, 1)


# speedup vs baseline: 1.5484x; 1.5484x over previous
import functools

import jax
import jax.numpy as jnp
from jax import lax
from jax.experimental import pallas as pl
from jax.experimental.pallas import tpu as pltpu
from jax.experimental.pallas import tpu_sc as plsc

NUM_ITEMS = 1000000
D = 64
B = 4096
L = 16

_info = plsc.get_sparse_core_info()
NC, NS = _info.num_cores, _info.num_subcores
NW = NC * NS
BPW = B // NW

_mesh = plsc.VectorSubcoreMesh(core_axis_name="c", subcore_axis_name="s")


def _sigmoid_v(x):
    return 1.0 / (1.0 + jnp.exp(-x))


def _hsum_splat(x):
    lanes = jnp.arange(L, dtype=jnp.int32)
    for stride in (8, 4, 2, 1):
        x = x + x.at[lanes ^ stride].get(mode="promise_in_bounds")
    return x


def _fetch_rows(idx_v, gmf_hbm, sdae_hbm, gmf_v, sdae_v, sem1, sem2):
    def chunk(ci, carry):
        idx16 = idx_v[pl.ds(ci * L, L)]
        cps = []
        for jj in range(L):
            r = idx16[jj]
            j = ci * L + jj
            cps.append(pltpu.async_copy(gmf_hbm.at[r], gmf_v.at[j], sem1))
            cps.append(pltpu.async_copy(sdae_hbm.at[r], sdae_v.at[j], sem2))
        for cp in cps:
            cp.wait()
        return carry

    lax.fori_loop(0, BPW // L, chunk, 0)


@functools.partial(
    pl.kernel,
    mesh=_mesh,
    out_type=jax.ShapeDtypeStruct((NW, D), jnp.float32),
    scratch_types=[
        pltpu.VMEM((BPW,), jnp.int32),
        pltpu.VMEM((BPW, D), jnp.float32),
        pltpu.VMEM((BPW, D), jnp.float32),
        pltpu.VMEM((D,), jnp.float32),
        pltpu.VMEM((L,), jnp.float32),
        pltpu.VMEM((D,), jnp.float32),
        pltpu.SemaphoreType.DMA,
        pltpu.SemaphoreType.DMA,
    ],
)
def _others_partials(idx_hbm, gmf_hbm, sdae_hbm, w_hbm, b_hbm, part_hbm,
                     idx_v, gmf_v, sdae_v, w_v, b_v, acc_v, sem1, sem2):
    wid = lax.axis_index("s") * NC + lax.axis_index("c")
    base = wid * BPW
    pltpu.sync_copy(idx_hbm.at[pl.ds(base, BPW)], idx_v)
    pltpu.sync_copy(w_hbm, w_v)
    pltpu.sync_copy(b_hbm, b_v)
    _fetch_rows(idx_v, gmf_hbm, sdae_hbm, gmf_v, sdae_v, sem1, sem2)

    w = [w_v[pl.ds(k * L, L)] for k in range(D // L)]
    bvec = b_v[...]

    def row(j, acc):
        g = [gmf_v[j, pl.ds(k * L, L)] for k in range(D // L)]
        t = g[0] * w[0]
        for k in range(1, D // L):
            t = t + g[k] * w[k]
        zv = _hsum_splat(t) + bvec
        dv = _sigmoid_v(zv)
        out = []
        for k in range(D // L):
            sd = sdae_v[j, pl.ds(k * L, L)]
            out.append(acc[k] + g[k] + dv * (sd - g[k]))
        return tuple(out)

    zero = jnp.zeros((L,), jnp.float32)
    acc = lax.fori_loop(0, BPW, row, (zero,) * (D // L))
    for k in range(D // L):
        acc_v[pl.ds(k * L, L)] = acc[k]
    pltpu.sync_copy(acc_v, part_hbm.at[wid])


@functools.partial(
    pl.kernel,
    mesh=_mesh,
    out_type=jax.ShapeDtypeStruct((B,), jnp.float32),
    scratch_types=[
        pltpu.VMEM((BPW,), jnp.int32),
        pltpu.VMEM((BPW, D), jnp.float32),
        pltpu.VMEM((BPW, D), jnp.float32),
        pltpu.VMEM((D,), jnp.float32),
        pltpu.VMEM((L,), jnp.float32),
        pltpu.VMEM((NW, D), jnp.float32),
        pltpu.VMEM((BPW,), jnp.float32),
        pltpu.SemaphoreType.DMA,
        pltpu.SemaphoreType.DMA,
    ],
)
def _target_dots(idx_hbm, gmf_hbm, sdae_hbm, w_hbm, b_hbm, part_hbm, out_hbm,
                 idx_v, gmf_v, sdae_v, w_v, b_v, p_v, out_v, sem1, sem2):
    wid = lax.axis_index("s") * NC + lax.axis_index("c")
    base = wid * BPW
    pltpu.sync_copy(idx_hbm.at[pl.ds(base, BPW)], idx_v)
    pltpu.sync_copy(w_hbm, w_v)
    pltpu.sync_copy(b_hbm, b_v)
    pltpu.sync_copy(part_hbm, p_v)
    _fetch_rows(idx_v, gmf_hbm, sdae_hbm, gmf_v, sdae_v, sem1, sem2)

    w = [w_v[pl.ds(k * L, L)] for k in range(D // L)]
    bvec = b_v[...]
    s = []
    for k in range(D // L):
        sk = p_v[0, pl.ds(k * L, L)]
        for n in range(1, NW):
            sk = sk + p_v[n, pl.ds(k * L, L)]
        s.append(sk)

    lanes = lax.iota(jnp.int32, L)

    def chunk(ci, carry):
        dotv = jnp.zeros((L,), jnp.float32)
        for jj in range(L):
            j = ci * L + jj
            g = [gmf_v[j, pl.ds(k * L, L)] for k in range(D // L)]
            t = g[0] * w[0]
            for k in range(1, D // L):
                t = t + g[k] * w[k]
            zv = _hsum_splat(t) + bvec
            dv = _sigmoid_v(zv)
            u = None
            for k in range(D // L):
                sd = sdae_v[j, pl.ds(k * L, L)]
                blend = g[k] + dv * (sd - g[k])
                u = blend * s[k] if u is None else u + blend * s[k]
            dotv = jnp.where(lanes == jj, _hsum_splat(u), dotv)
        out_v[pl.ds(ci * L, L)] = _sigmoid_v(dotv)
        return carry

    lax.fori_loop(0, BPW // L, chunk, 0)
    pltpu.sync_copy(out_v, out_hbm.at[pl.ds(base, BPW)])


def kernel(item_indices1, item_indices2, table_gmf, table_sdae, delta_W, delta_b):
    w = delta_W.reshape(D).astype(jnp.float32)
    b = jnp.broadcast_to(delta_b.reshape(1), (L,)).astype(jnp.float32)
    partials = _others_partials(item_indices2.astype(jnp.int32),
                                table_gmf, table_sdae, w, b)
    dots = _target_dots(item_indices1.astype(jnp.int32),
                        table_gmf, table_sdae, w, b, partials)
    return dots.reshape(B, 1)


# speedup vs baseline: 5.1318x; 3.3143x over previous
import functools

import jax
import jax.numpy as jnp
from jax import lax
from jax.experimental import pallas as pl
from jax.experimental.pallas import tpu as pltpu
from jax.experimental.pallas import tpu_sc as plsc

NUM_ITEMS = 1000000
D = 64
B = 4096
L = 16
KG = D // L
NBUF = 4

_info = plsc.get_sparse_core_info()
NC, NS = _info.num_cores, _info.num_subcores
NW = NC * NS
BPW = B // NW

_mesh = plsc.VectorSubcoreMesh(core_axis_name="c", subcore_axis_name="s")


def _sigmoid_v(x):
    return 1.0 / (1.0 + jnp.exp(-x))


def _hsum_splat(x):
    lanes = jnp.arange(L, dtype=jnp.int32)
    for stride in (8, 4, 2, 1):
        x = x + x.at[lanes ^ stride].get(mode="promise_in_bounds")
    return x


def _splat_at(ref, j):
    return plsc.load_gather(ref, [jnp.full((L,), j, jnp.int32)])


def _col_issue(gmf_hbm, sdae_hbm, g_buf, sd_buf, cols_v, j, b, semg, sems):
    c = _splat_at(cols_v, j)[0]
    start = c * 128
    pltpu.async_copy(gmf_hbm.at[:, pl.ds(start, 128)], g_buf.at[b], semg)
    pltpu.async_copy(sdae_hbm.at[:, pl.ds(start, 128)], sd_buf.at[b], sems)


def _col_drain(gmf_hbm, sdae_hbm, g_buf, sd_buf, b, semg, sems):
    pltpu.make_async_copy(
        gmf_hbm.at[:, pl.ds(0, 128)], g_buf.at[b], semg).wait()
    pltpu.make_async_copy(
        sdae_hbm.at[:, pl.ds(0, 128)], sd_buf.at[b], sems).wait()


def _extract_blend(g_buf, sd_buf, b, lanes_v, j, w, bvec):
    lv = _splat_at(lanes_v, j)
    g = []
    sd = []
    for t in range(KG):
        fidx = jnp.arange(L, dtype=jnp.int32) + t * L
        g.append(plsc.load_gather(g_buf.at[b], [fidx, lv]))
        sd.append(plsc.load_gather(sd_buf.at[b], [fidx, lv]))
    z = None
    for t in range(KG):
        zt = g[t] * w[t]
        z = zt if z is None else z + zt
    d = _sigmoid_v(_hsum_splat(z) + bvec)
    return [g[t] + d * (sd[t] - g[t]) for t in range(KG)]


@functools.partial(
    pl.kernel,
    mesh=_mesh,
    compiler_params=pltpu.CompilerParams(needs_layout_passes=False),
    out_type=(
        jax.ShapeDtypeStruct((NW, D), jnp.float32),
        jax.ShapeDtypeStruct((B, D), jnp.float32),
    ),
    scratch_types=[
        pltpu.VMEM((BPW,), jnp.int32),
        pltpu.VMEM((BPW,), jnp.int32),
        pltpu.VMEM((BPW,), jnp.int32),
        pltpu.VMEM((BPW,), jnp.int32),
        pltpu.VMEM((NBUF, D, 128), jnp.float32),
        pltpu.VMEM((NBUF, D, 128), jnp.float32),
        pltpu.VMEM((BPW, D), jnp.float32),
        pltpu.VMEM((D,), jnp.float32),
        pltpu.VMEM((L,), jnp.float32),
        pltpu.VMEM((D,), jnp.float32),
        pltpu.SemaphoreType.DMA,
        pltpu.SemaphoreType.DMA,
    ],
)
def _scan_blend(idx1_hbm, idx2_hbm, gmf_hbm, sdae_hbm, w_hbm, b_hbm,
                part_hbm, stage_hbm,
                cols1_v, lanes1_v, cols2_v, lanes2_v, g_buf, sd_buf,
                stage_v, w_v, b_v, po_v, semg, sems):
    wid = lax.axis_index("s") * NC + lax.axis_index("c")
    base = wid * BPW
    pltpu.sync_copy(w_hbm, w_v)
    pltpu.sync_copy(b_hbm, b_v)
    pltpu.sync_copy(idx1_hbm.at[pl.ds(base, BPW)], cols1_v)
    pltpu.sync_copy(idx2_hbm.at[pl.ds(base, BPW)], cols2_v)
    for ci in range(BPW // L):
        sl = pl.ds(ci * L, L)
        for cv, lv in ((cols1_v, lanes1_v), (cols2_v, lanes2_v)):
            idx16 = cv[sl]
            lv[sl] = jnp.bitwise_and(idx16, 127)
            cv[sl] = jnp.right_shift(idx16, 7)

    w = [w_v[pl.ds(t * L, L)] for t in range(KG)]
    bvec = b_v[...]
    zero = jnp.zeros((L,), jnp.float32)

    for b in range(NBUF):
        _col_issue(gmf_hbm, sdae_hbm, g_buf, sd_buf, cols1_v,
                   jnp.int32(b), b, semg, sems)

    def tgt_group(g, carry):
        for b in range(NBUF):
            j = g * NBUF + b
            _col_drain(gmf_hbm, sdae_hbm, g_buf, sd_buf, b, semg, sems)
            blend = _extract_blend(g_buf, sd_buf, b, lanes1_v, j, w, bvec)
            for t in range(KG):
                stage_v[j, pl.ds(t * L, L)] = blend[t]
            nj = j + NBUF

            @pl.when(nj < BPW)
            def _():
                _col_issue(gmf_hbm, sdae_hbm, g_buf, sd_buf, cols1_v,
                           nj, b, semg, sems)
        return carry

    lax.fori_loop(0, BPW // NBUF, tgt_group, 0)
    pltpu.sync_copy(stage_v, stage_hbm.at[pl.ds(base, BPW)])

    for b in range(NBUF):
        _col_issue(gmf_hbm, sdae_hbm, g_buf, sd_buf, cols2_v,
                   jnp.int32(b), b, semg, sems)

    def oth_group(g, acc):
        acc = list(acc)
        for b in range(NBUF):
            j = g * NBUF + b
            _col_drain(gmf_hbm, sdae_hbm, g_buf, sd_buf, b, semg, sems)
            blend = _extract_blend(g_buf, sd_buf, b, lanes2_v, j, w, bvec)
            for t in range(KG):
                acc[t] = acc[t] + blend[t]
            nj = j + NBUF

            @pl.when(nj < BPW)
            def _():
                _col_issue(gmf_hbm, sdae_hbm, g_buf, sd_buf, cols2_v,
                           nj, b, semg, sems)
        return tuple(acc)

    acc = lax.fori_loop(0, BPW // NBUF, oth_group, (zero,) * KG)

    for t in range(KG):
        po_v[pl.ds(t * L, L)] = acc[t]
    pltpu.sync_copy(po_v, part_hbm.at[wid])


@functools.partial(
    pl.kernel,
    mesh=_mesh,
    out_type=jax.ShapeDtypeStruct((B,), jnp.float32),
    scratch_types=[
        pltpu.VMEM((BPW, D), jnp.float32),
        pltpu.VMEM((NW, D), jnp.float32),
        pltpu.VMEM((BPW,), jnp.float32),
    ],
)
def _dots(part_hbm, stage_hbm, out_hbm, st_v, p_v, out_v):
    wid = lax.axis_index("s") * NC + lax.axis_index("c")
    base = wid * BPW
    pltpu.sync_copy(stage_hbm.at[pl.ds(base, BPW)], st_v)
    pltpu.sync_copy(part_hbm, p_v)

    svec = []
    for t in range(KG):
        sk = p_v[0, pl.ds(t * L, L)]
        for n in range(1, NW):
            sk = sk + p_v[n, pl.ds(t * L, L)]
        svec.append(sk)

    lanes = lax.iota(jnp.int32, L)

    def chunk(ci, carry):
        dotv = jnp.zeros((L,), jnp.float32)
        for jj in range(L):
            j = ci * L + jj
            u = None
            for t in range(KG):
                ut = st_v[j, pl.ds(t * L, L)] * svec[t]
                u = ut if u is None else u + ut
            dotv = jnp.where(lanes == jj, _hsum_splat(u), dotv)
        out_v[pl.ds(ci * L, L)] = _sigmoid_v(dotv)
        return carry

    lax.fori_loop(0, BPW // L, chunk, 0)
    pltpu.sync_copy(out_v, out_hbm.at[pl.ds(base, BPW)])


def kernel(item_indices1, item_indices2, table_gmf, table_sdae, delta_W, delta_b):
    gmf_t = table_gmf.T
    sdae_t = table_sdae.T
    w = delta_W.reshape(D).astype(jnp.float32)
    b = jnp.broadcast_to(delta_b.reshape(1), (L,)).astype(jnp.float32)
    partials, staged = _scan_blend(item_indices1.astype(jnp.int32),
                                   item_indices2.astype(jnp.int32),
                                   gmf_t, sdae_t, w, b)
    dots = _dots(partials, staged)
    return dots.reshape(B, 1)


# speedup vs baseline: 5.2026x; 1.0138x over previous
import functools

import jax
import jax.numpy as jnp
from jax import lax
from jax.experimental import pallas as pl
from jax.experimental.pallas import tpu as pltpu
from jax.experimental.pallas import tpu_sc as plsc

NUM_ITEMS = 1000000
D = 64
B = 4096
L = 16
KG = D // L
NBUF = 6

_info = plsc.get_sparse_core_info()
NC, NS = _info.num_cores, _info.num_subcores
NW = NC * NS
BPW = B // NW
NGRP = BPW // NBUF
NTAIL = BPW - NGRP * NBUF

_mesh = plsc.VectorSubcoreMesh(core_axis_name="c", subcore_axis_name="s")


def _sigmoid_v(x):
    return 1.0 / (1.0 + jnp.exp(-x))


def _hsum_splat(x):
    lanes = jnp.arange(L, dtype=jnp.int32)
    for stride in (8, 4, 2, 1):
        x = x + x.at[lanes ^ stride].get(mode="promise_in_bounds")
    return x


def _splat_at(ref, j):
    return plsc.load_gather(ref, [jnp.full((L,), j, jnp.int32)])


def _col_issue(gmf_hbm, sdae_hbm, g_buf, sd_buf, cols_v, j, b, semg, sems):
    c = _splat_at(cols_v, j)[0]
    start = c * 128
    pltpu.async_copy(gmf_hbm.at[:, pl.ds(start, 128)], g_buf.at[b], semg)
    pltpu.async_copy(sdae_hbm.at[:, pl.ds(start, 128)], sd_buf.at[b], sems)


def _col_drain(gmf_hbm, sdae_hbm, g_buf, sd_buf, b, semg, sems):
    pltpu.make_async_copy(
        gmf_hbm.at[:, pl.ds(0, 128)], g_buf.at[b], semg).wait()
    pltpu.make_async_copy(
        sdae_hbm.at[:, pl.ds(0, 128)], sd_buf.at[b], sems).wait()


def _extract_blend(g_buf, sd_buf, b, lanes_v, j, w, bvec):
    lv = _splat_at(lanes_v, j)
    g = []
    sd = []
    for t in range(KG):
        fidx = jnp.arange(L, dtype=jnp.int32) + t * L
        g.append(plsc.load_gather(g_buf.at[b], [fidx, lv]))
        sd.append(plsc.load_gather(sd_buf.at[b], [fidx, lv]))
    z = None
    for t in range(KG):
        zt = g[t] * w[t]
        z = zt if z is None else z + zt
    d = _sigmoid_v(_hsum_splat(z) + bvec)
    return [g[t] + d * (sd[t] - g[t]) for t in range(KG)]


@functools.partial(
    pl.kernel,
    mesh=_mesh,
    compiler_params=pltpu.CompilerParams(needs_layout_passes=False),
    out_type=(
        jax.ShapeDtypeStruct((NW, D), jnp.float32),
        jax.ShapeDtypeStruct((B, D), jnp.float32),
    ),
    scratch_types=[
        pltpu.VMEM((BPW,), jnp.int32),
        pltpu.VMEM((BPW,), jnp.int32),
        pltpu.VMEM((BPW,), jnp.int32),
        pltpu.VMEM((BPW,), jnp.int32),
        pltpu.VMEM((NBUF, D, 128), jnp.float32),
        pltpu.VMEM((NBUF, D, 128), jnp.float32),
        pltpu.VMEM((BPW, D), jnp.float32),
        pltpu.VMEM((D,), jnp.float32),
        pltpu.VMEM((L,), jnp.float32),
        pltpu.VMEM((D,), jnp.float32),
        pltpu.SemaphoreType.DMA,
        pltpu.SemaphoreType.DMA,
    ],
)
def _scan_blend(idx1_hbm, idx2_hbm, gmf_hbm, sdae_hbm, w_hbm, b_hbm,
                part_hbm, stage_hbm,
                cols1_v, lanes1_v, cols2_v, lanes2_v, g_buf, sd_buf,
                stage_v, w_v, b_v, po_v, semg, sems):
    wid = lax.axis_index("s") * NC + lax.axis_index("c")
    base = wid * BPW
    pltpu.sync_copy(w_hbm, w_v)
    pltpu.sync_copy(b_hbm, b_v)
    pltpu.sync_copy(idx1_hbm.at[pl.ds(base, BPW)], cols1_v)
    pltpu.sync_copy(idx2_hbm.at[pl.ds(base, BPW)], cols2_v)
    for ci in range(BPW // L):
        sl = pl.ds(ci * L, L)
        for cv, lv in ((cols1_v, lanes1_v), (cols2_v, lanes2_v)):
            idx16 = cv[sl]
            lv[sl] = jnp.bitwise_and(idx16, 127)
            cv[sl] = jnp.right_shift(idx16, 7)

    w = [w_v[pl.ds(t * L, L)] for t in range(KG)]
    bvec = b_v[...]
    zero = jnp.zeros((L,), jnp.float32)

    def run_phase(cols_v, lanes_v, sink):
        for b in range(NBUF):
            _col_issue(gmf_hbm, sdae_hbm, g_buf, sd_buf, cols_v,
                       jnp.int32(b), b, semg, sems)

        def slot(j, b):
            _col_drain(gmf_hbm, sdae_hbm, g_buf, sd_buf, b, semg, sems)
            blend = _extract_blend(g_buf, sd_buf, b, lanes_v, j, w, bvec)
            sink(j, blend)
            nj = j + NBUF

            @pl.when(nj < BPW)
            def _():
                _col_issue(gmf_hbm, sdae_hbm, g_buf, sd_buf, cols_v,
                           nj, b, semg, sems)

        def group(g, carry):
            for b in range(NBUF):
                slot(g * NBUF + b, b)
            return carry

        lax.fori_loop(0, NGRP, group, 0)
        for b in range(NTAIL):
            slot(NGRP * NBUF + b, b)

    def tgt_sink(j, blend):
        for t in range(KG):
            stage_v[j, pl.ds(t * L, L)] = blend[t]

    run_phase(cols1_v, lanes1_v, tgt_sink)
    pltpu.sync_copy(stage_v, stage_hbm.at[pl.ds(base, BPW)])

    for t in range(KG):
        po_v[pl.ds(t * L, L)] = zero

    def oth_sink(j, blend):
        for t in range(KG):
            plsc.addupdate(po_v.at[pl.ds(t * L, L)], blend[t])

    run_phase(cols2_v, lanes2_v, oth_sink)
    pltpu.sync_copy(po_v, part_hbm.at[wid])


@functools.partial(
    pl.kernel,
    mesh=_mesh,
    out_type=jax.ShapeDtypeStruct((B,), jnp.float32),
    scratch_types=[
        pltpu.VMEM((BPW, D), jnp.float32),
        pltpu.VMEM((NW, D), jnp.float32),
        pltpu.VMEM((BPW,), jnp.float32),
    ],
)
def _dots(part_hbm, stage_hbm, out_hbm, st_v, p_v, out_v):
    wid = lax.axis_index("s") * NC + lax.axis_index("c")
    base = wid * BPW
    pltpu.sync_copy(stage_hbm.at[pl.ds(base, BPW)], st_v)
    pltpu.sync_copy(part_hbm, p_v)

    svec = []
    for t in range(KG):
        sk = p_v[0, pl.ds(t * L, L)]
        for n in range(1, NW):
            sk = sk + p_v[n, pl.ds(t * L, L)]
        svec.append(sk)

    lanes = lax.iota(jnp.int32, L)

    def chunk(ci, carry):
        dotv = jnp.zeros((L,), jnp.float32)
        for jj in range(L):
            j = ci * L + jj
            u = None
            for t in range(KG):
                ut = st_v[j, pl.ds(t * L, L)] * svec[t]
                u = ut if u is None else u + ut
            dotv = jnp.where(lanes == jj, _hsum_splat(u), dotv)
        out_v[pl.ds(ci * L, L)] = _sigmoid_v(dotv)
        return carry

    lax.fori_loop(0, BPW // L, chunk, 0)
    pltpu.sync_copy(out_v, out_hbm.at[pl.ds(base, BPW)])


def kernel(item_indices1, item_indices2, table_gmf, table_sdae, delta_W, delta_b):
    gmf_t = table_gmf.T
    sdae_t = table_sdae.T
    w = delta_W.reshape(D).astype(jnp.float32)
    b = jnp.broadcast_to(delta_b.reshape(1), (L,)).astype(jnp.float32)
    partials, staged = _scan_blend(item_indices1.astype(jnp.int32),
                                   item_indices2.astype(jnp.int32),
                                   gmf_t, sdae_t, w, b)
    dots = _dots(partials, staged)
    return dots.reshape(B, 1)
